# baseline (speedup 1.0000x reference)
import functools

import jax
import jax.numpy as jnp
import numpy as np
from jax import lax
from jax.experimental import pallas as pl
from jax.experimental.pallas import tpu as pltpu
from jax.experimental.pallas import tpu_sc as plsc

N = 4096
K = 32
D = 128
H = 128
NW = 32
NPW = N // NW
GROUP = 16
NGROUPS = NPW // GROUP
ROWS = GROUP * K
CHUNK = 128
NCHUNK = NPW * K // CHUNK


def _leaky(x):
    return jnp.where(x >= 0, x, 0.3 * x)


def _project_body(e_ref, wq_ref, bqc_ref, wk_ref, bk_ref, wv_ref, bv_ref,
                  eqt_ref, ek_ref, ev_ref):
    e = e_ref[...]
    ek_ref[...] = _leaky(
        jnp.dot(e, wk_ref[...], preferred_element_type=jnp.float32)
        + bk_ref[...]).astype(jnp.bfloat16)
    ev_ref[...] = _leaky(
        jnp.dot(e, wv_ref[...], preferred_element_type=jnp.float32)
        + bv_ref[...]).astype(jnp.bfloat16)
    eqt = lax.dot_general(wq_ref[...], e, (((0,), (1,)), ((), ())),
                          preferred_element_type=jnp.float32)
    eqt_ref[0] = _leaky(eqt + bqc_ref[...])


def _project(e, WQ, bQc, WK, bK2, WV, bV2):
    return pl.pallas_call(
        _project_body,
        grid=(NW,),
        in_specs=[
            pl.BlockSpec((NPW, D), lambda g: (g, 0)),
            pl.BlockSpec((D, H), lambda g: (0, 0)),
            pl.BlockSpec((H, 1), lambda g: (0, 0)),
            pl.BlockSpec((D, H), lambda g: (0, 0)),
            pl.BlockSpec((1, H), lambda g: (0, 0)),
            pl.BlockSpec((D, H), lambda g: (0, 0)),
            pl.BlockSpec((1, H), lambda g: (0, 0)),
        ],
        out_specs=[
            pl.BlockSpec((1, H, NPW), lambda g: (g, 0, 0)),
            pl.BlockSpec((NPW, H), lambda g: (g, 0)),
            pl.BlockSpec((NPW, H), lambda g: (g, 0)),
        ],
        out_shape=[
            jax.ShapeDtypeStruct((NW, H, NPW), jnp.float32),
            jax.ShapeDtypeStruct((N, H), jnp.bfloat16),
            jax.ShapeDtypeStruct((N, H), jnp.bfloat16),
        ],
    )(e, WQ, bQc, WK, bK2, WV, bV2)


NOCT = 4
KO = K // NOCT
_MASK_HI = np.uint32(0xFFFF0000)


def _unpack_lo(w):
    u = lax.bitcast_convert_type(w, jnp.uint32)
    return lax.bitcast_convert_type(u << 16, jnp.float32)


def _unpack_hi(w):
    u = lax.bitcast_convert_type(w, jnp.uint32)
    return lax.bitcast_convert_type(u & _MASK_HI, jnp.float32)


def _sc_attend_body(ns_hbm, eqt_hbm, evek_hbm, outt_hbm,
                    idx_v, rows_v, eqt_v, eqtrot_v, ptrot_v, sem):
    wid = lax.axis_index("s") * 2 + lax.axis_index("c")
    pltpu.sync_copy(ns_hbm.at[wid], idx_v)
    pltpu.sync_copy(eqt_hbm.at[wid], eqt_v)

    lanes = lax.iota(jnp.int32, 16)

    def fire(g, c):
        return pltpu.async_copy(
            evek_hbm.at[idx_v.at[NOCT * g + c]],
            rows_v.at[pl.ds(c * CHUNK, CHUNK)],
            sem)

    def rot_h(h, carry):
        hrot = (h + lanes) & (H - 1)
        for s8 in range(NPW // 16):
            col = jnp.full((16,), s8 * 16, jnp.int32) + lanes
            eqtrot_v[h, pl.ds(s8 * 16, 16)] = plsc.load_gather(
                eqt_v, [hrot, col])
        return carry

    lax.fori_loop(0, H, rot_h, 0)

    pending = [fire(0, c) for c in range(NOCT)]

    for g in range(NGROUPS):
        g16 = g * GROUP

        s = []
        for c in range(NOCT):
            pending[c].wait()

            def score_h(h, sc, c=c):
                colrot = (jnp.full((16,), 0, jnp.int32) + h + lanes) & (H - 1)
                eqt = eqtrot_v[h, pl.ds(g16, GROUP)]
                return tuple(
                    sc[kk] + eqt * _unpack_lo(plsc.load_gather(
                        rows_v, [lanes * KO + (c * CHUNK + kk), colrot]))
                    for kk in range(KO)
                )

            s0 = tuple(jnp.zeros((16,), jnp.float32) for _ in range(KO))
            s.extend(lax.fori_loop(0, H, score_h, s0))

        m = s[0]
        for k in range(1, K):
            m = jnp.maximum(m, s[k])
        e = [jnp.exp(s[k] - m) for k in range(K)]
        den = e[0]
        for k in range(1, K):
            den = den + e[k]
        inv = 1.0 / den
        a = [e[k] * inv for k in range(K)]

        for c in range(NOCT):
            ac = a[c * KO:(c + 1) * KO]

            def pool_h(h, carry, c=c, ac=ac):
                colrot = (jnp.full((16,), 0, jnp.int32) + h + lanes) & (H - 1)
                acc = ac[0] * _unpack_hi(plsc.load_gather(
                    rows_v, [lanes * KO + c * CHUNK, colrot]))
                for kk in range(1, KO):
                    acc = acc + ac[kk] * _unpack_hi(plsc.load_gather(
                        rows_v, [lanes * KO + (c * CHUNK + kk), colrot]))
                if c == 0:
                    ptrot_v[h, pl.ds(g16, GROUP)] = acc
                else:
                    ptrot_v[h, pl.ds(g16, GROUP)] += acc
                return carry

            lax.fori_loop(0, H, pool_h, 0)
            if g + 1 < NGROUPS:
                pending[c] = fire(g + 1, c)

    def unrot_h(h, carry):
        hrot = (h - lanes) & (H - 1)
        for s8 in range(NPW // 16):
            col = jnp.full((16,), s8 * 16, jnp.int32) + lanes
            eqt_v[h, pl.ds(s8 * 16, 16)] = plsc.load_gather(
                ptrot_v, [hrot, col])
        return carry

    lax.fori_loop(0, H, unrot_h, 0)

    pltpu.sync_copy(eqt_v, outt_hbm.at[wid])


def _sc_attend(ns_r, eqt_blocks, evek):
    mesh = plsc.VectorSubcoreMesh(core_axis_name="c", subcore_axis_name="s")
    run = functools.partial(
        pl.kernel,
        mesh=mesh,
        compiler_params=pltpu.CompilerParams(needs_layout_passes=False),
        out_type=jax.ShapeDtypeStruct((NW, H, NPW), jnp.float32),
        scratch_types=[
            pltpu.VMEM((NCHUNK, CHUNK), jnp.int32),
            pltpu.VMEM((ROWS, H), jnp.float32),
            pltpu.VMEM((H, NPW), jnp.float32),
            pltpu.VMEM((H, NPW), jnp.float32),
            pltpu.VMEM((H, NPW), jnp.float32),
            pltpu.SemaphoreType.DMA,
        ],
    )(_sc_attend_body)
    return run(ns_r, eqt_blocks, evek)


def _post_body(e_ref, pt_ref, w1a_ref, w1b_ref, b1_ref,
               gamma_ref, beta_ref, mm_ref, mv_ref, out_ref):
    e = e_ref[...]
    hidden = _leaky(
        jnp.dot(e, w1a_ref[...], preferred_element_type=jnp.float32)
        + lax.dot_general(pt_ref[0], w1b_ref[...], (((0,), (0,)), ((), ())),
                          preferred_element_type=jnp.float32)
        + b1_ref[...])
    nrm = jnp.sqrt(jnp.sum(hidden * hidden, axis=1, keepdims=True))
    normalized = hidden / (nrm + 1e-6)
    out_ref[...] = (gamma_ref[...] * (normalized - mm_ref[...])
                    / jnp.sqrt(mv_ref[...] + 1e-3) + beta_ref[...])


def _post(e, pooledt, W1a, W1b, b12, gamma2, beta2, mm2, mv2):
    return pl.pallas_call(
        _post_body,
        grid=(NW,),
        in_specs=[
            pl.BlockSpec((NPW, D), lambda g: (g, 0)),
            pl.BlockSpec((1, H, NPW), lambda g: (g, 0, 0)),
            pl.BlockSpec((D, H), lambda g: (0, 0)),
            pl.BlockSpec((H, H), lambda g: (0, 0)),
            pl.BlockSpec((1, H), lambda g: (0, 0)),
            pl.BlockSpec((1, H), lambda g: (0, 0)),
            pl.BlockSpec((1, H), lambda g: (0, 0)),
            pl.BlockSpec((1, H), lambda g: (0, 0)),
            pl.BlockSpec((1, H), lambda g: (0, 0)),
        ],
        out_specs=pl.BlockSpec((NPW, H), lambda g: (g, 0)),
        out_shape=jax.ShapeDtypeStruct((N, H), jnp.float32),
    )(e, pooledt, W1a, W1b, b12, gamma2, beta2, mm2, mv2)


def kernel(embeddings, weights, neighbor_set, WQ, bQ, WK, bK, WV, bV, W1, b1,
           gamma, beta, moving_mean, moving_var):
    e = embeddings[0]
    ns_r = (neighbor_set[0]
            .reshape(NW, NGROUPS, GROUP, NOCT, KO)
            .transpose(0, 1, 3, 2, 4)
            .reshape(NW, NCHUNK, CHUNK))

    eqt_blocks, ek, ev = _project(
        e, WQ, bQ.reshape(H, 1), WK, bK.reshape(1, H), WV, bV.reshape(1, H))
    evek = lax.bitcast_convert_type(
        jnp.stack([ev, ek], axis=-1), jnp.float32)

    pooledt = _sc_attend(ns_r, eqt_blocks, evek)

    out = _post(
        e, pooledt, W1[:D], W1[D:], b1.reshape(1, H),
        gamma.reshape(1, H), beta.reshape(1, H),
        moving_mean.reshape(1, H), moving_var.reshape(1, H))
    return out.reshape(1, N, H)


# speedup vs baseline: 1.1843x; 1.1843x over previous
import functools

import jax
import jax.numpy as jnp
import numpy as np
from jax import lax
from jax.experimental import pallas as pl
from jax.experimental.pallas import tpu as pltpu
from jax.experimental.pallas import tpu_sc as plsc

N = 4096
K = 32
D = 128
H = 128
NW = 32
NPW = N // NW
GROUP = 16
NGROUPS = NPW // GROUP
ROWS = GROUP * K
CHUNK = 128
NCHUNK = NPW * K // CHUNK


def _leaky(x):
    return jnp.where(x >= 0, x, 0.3 * x)


def _project_body(e_ref, wq_ref, bqc_ref, wk_ref, bk_ref, wv_ref, bv_ref,
                  eqt_ref, ek_ref, ev_ref):
    e = e_ref[...]
    ek_ref[...] = _leaky(
        jnp.dot(e, wk_ref[...], preferred_element_type=jnp.float32)
        + bk_ref[...]).astype(jnp.bfloat16)
    ev_ref[...] = _leaky(
        jnp.dot(e, wv_ref[...], preferred_element_type=jnp.float32)
        + bv_ref[...]).astype(jnp.bfloat16)
    eqt = lax.dot_general(wq_ref[...], e, (((0,), (1,)), ((), ())),
                          preferred_element_type=jnp.float32)
    eqt_ref[0] = _leaky(eqt + bqc_ref[...])


def _project(e, WQ, bQc, WK, bK2, WV, bV2):
    return pl.pallas_call(
        _project_body,
        grid=(NW,),
        in_specs=[
            pl.BlockSpec((NPW, D), lambda g: (g, 0)),
            pl.BlockSpec((D, H), lambda g: (0, 0)),
            pl.BlockSpec((H, 1), lambda g: (0, 0)),
            pl.BlockSpec((D, H), lambda g: (0, 0)),
            pl.BlockSpec((1, H), lambda g: (0, 0)),
            pl.BlockSpec((D, H), lambda g: (0, 0)),
            pl.BlockSpec((1, H), lambda g: (0, 0)),
        ],
        out_specs=[
            pl.BlockSpec((1, H, NPW), lambda g: (g, 0, 0)),
            pl.BlockSpec((NPW, H), lambda g: (g, 0)),
            pl.BlockSpec((NPW, H), lambda g: (g, 0)),
        ],
        out_shape=[
            jax.ShapeDtypeStruct((NW, H, NPW), jnp.float32),
            jax.ShapeDtypeStruct((N, H), jnp.bfloat16),
            jax.ShapeDtypeStruct((N, H), jnp.bfloat16),
        ],
    )(e, WQ, bQc, WK, bK2, WV, bV2)


NOCT = 4
KO = K // NOCT
_MASK_HI = np.uint32(0xFFFF0000)


def _unpack_lo(w):
    u = lax.bitcast_convert_type(w, jnp.uint32)
    return lax.bitcast_convert_type(u << 16, jnp.float32)


def _unpack_hi(w):
    u = lax.bitcast_convert_type(w, jnp.uint32)
    return lax.bitcast_convert_type(u & _MASK_HI, jnp.float32)


def _sc_attend_body(ns_hbm, eqt_hbm, evek_hbm, outt_hbm,
                    idx_v, rows_v, eqt_v, eqtrot_v, ptrot_v, sem):
    wid = lax.axis_index("s") * 2 + lax.axis_index("c")
    pltpu.sync_copy(ns_hbm.at[wid], idx_v)
    pltpu.sync_copy(eqt_hbm.at[wid], eqt_v)

    lanes = lax.iota(jnp.int32, 16)

    def fire(g, c):
        return pltpu.async_copy(
            evek_hbm.at[idx_v.at[NOCT * g + c]],
            rows_v.at[pl.ds(c * CHUNK, CHUNK)],
            sem)

    def rot_h(h, carry):
        hrot = (h + lanes) & (H - 1)
        for s8 in range(NPW // 16):
            col = jnp.full((16,), s8 * 16, jnp.int32) + lanes
            eqtrot_v[h, pl.ds(s8 * 16, 16)] = plsc.load_gather(
                eqt_v, [hrot, col])
        return carry

    lax.fori_loop(0, H, rot_h, 0)

    pending = [fire(0, c) for c in range(NOCT)]

    for g in range(NGROUPS):
        g16 = g * GROUP

        s = []
        for c in range(NOCT):
            pending[c].wait()

            def score_h(i, carry, c=c):
                cr0 = carry[0]
                cr1 = (cr0 + 1) & (H - 1)
                sc = carry[1:]
                eqt0 = eqtrot_v[2 * i, pl.ds(g16, GROUP)]
                eqt1 = eqtrot_v[2 * i + 1, pl.ds(g16, GROUP)]
                out = []
                for kk in range(KO):
                    row = lanes * KO + (c * CHUNK + kk)
                    v0 = eqt0 * _unpack_lo(plsc.load_gather(rows_v, [row, cr0]))
                    v1 = eqt1 * _unpack_lo(plsc.load_gather(rows_v, [row, cr1]))
                    out.append(sc[kk] + (v0 + v1))
                return ((cr1 + 1) & (H - 1),) + tuple(out)

            s0 = (lanes,) + tuple(jnp.zeros((16,), jnp.float32)
                                  for _ in range(KO))
            s.extend(lax.fori_loop(0, H // 2, score_h, s0)[1:])

        m = s[0]
        for k in range(1, K):
            m = jnp.maximum(m, s[k])
        e = [jnp.exp(s[k] - m) for k in range(K)]
        den = e[0]
        for k in range(1, K):
            den = den + e[k]
        inv = 1.0 / den
        a = [e[k] * inv for k in range(K)]

        for c in range(NOCT):
            ac = a[c * KO:(c + 1) * KO]

            def pool_h(i, cr0, c=c, ac=ac):
                cr1 = (cr0 + 1) & (H - 1)
                acc0 = jnp.zeros((16,), jnp.float32)
                acc1 = jnp.zeros((16,), jnp.float32)
                for kk in range(KO):
                    row = lanes * KO + (c * CHUNK + kk)
                    acc0 = acc0 + ac[kk] * _unpack_hi(
                        plsc.load_gather(rows_v, [row, cr0]))
                    acc1 = acc1 + ac[kk] * _unpack_hi(
                        plsc.load_gather(rows_v, [row, cr1]))
                if c == 0:
                    ptrot_v[2 * i, pl.ds(g16, GROUP)] = acc0
                    ptrot_v[2 * i + 1, pl.ds(g16, GROUP)] = acc1
                else:
                    ptrot_v[2 * i, pl.ds(g16, GROUP)] += acc0
                    ptrot_v[2 * i + 1, pl.ds(g16, GROUP)] += acc1
                return (cr1 + 1) & (H - 1)

            lax.fori_loop(0, H // 2, pool_h, lanes)
            if g + 1 < NGROUPS:
                pending[c] = fire(g + 1, c)

    def unrot_h(h, carry):
        hrot = (h - lanes) & (H - 1)
        for s8 in range(NPW // 16):
            col = jnp.full((16,), s8 * 16, jnp.int32) + lanes
            eqt_v[h, pl.ds(s8 * 16, 16)] = plsc.load_gather(
                ptrot_v, [hrot, col])
        return carry

    lax.fori_loop(0, H, unrot_h, 0)

    pltpu.sync_copy(eqt_v, outt_hbm.at[wid])


def _sc_attend(ns_r, eqt_blocks, evek):
    mesh = plsc.VectorSubcoreMesh(core_axis_name="c", subcore_axis_name="s")
    run = functools.partial(
        pl.kernel,
        mesh=mesh,
        compiler_params=pltpu.CompilerParams(needs_layout_passes=False),
        out_type=jax.ShapeDtypeStruct((NW, H, NPW), jnp.float32),
        scratch_types=[
            pltpu.VMEM((NCHUNK, CHUNK), jnp.int32),
            pltpu.VMEM((ROWS, H), jnp.float32),
            pltpu.VMEM((H, NPW), jnp.float32),
            pltpu.VMEM((H, NPW), jnp.float32),
            pltpu.VMEM((H, NPW), jnp.float32),
            pltpu.SemaphoreType.DMA,
        ],
    )(_sc_attend_body)
    return run(ns_r, eqt_blocks, evek)


def _post_body(e_ref, pt_ref, w1a_ref, w1b_ref, b1_ref,
               gamma_ref, beta_ref, mm_ref, mv_ref, out_ref):
    e = e_ref[...]
    hidden = _leaky(
        jnp.dot(e, w1a_ref[...], preferred_element_type=jnp.float32)
        + lax.dot_general(pt_ref[0], w1b_ref[...], (((0,), (0,)), ((), ())),
                          preferred_element_type=jnp.float32)
        + b1_ref[...])
    nrm = jnp.sqrt(jnp.sum(hidden * hidden, axis=1, keepdims=True))
    normalized = hidden / (nrm + 1e-6)
    out_ref[...] = (gamma_ref[...] * (normalized - mm_ref[...])
                    / jnp.sqrt(mv_ref[...] + 1e-3) + beta_ref[...])


def _post(e, pooledt, W1a, W1b, b12, gamma2, beta2, mm2, mv2):
    return pl.pallas_call(
        _post_body,
        grid=(NW,),
        in_specs=[
            pl.BlockSpec((NPW, D), lambda g: (g, 0)),
            pl.BlockSpec((1, H, NPW), lambda g: (g, 0, 0)),
            pl.BlockSpec((D, H), lambda g: (0, 0)),
            pl.BlockSpec((H, H), lambda g: (0, 0)),
            pl.BlockSpec((1, H), lambda g: (0, 0)),
            pl.BlockSpec((1, H), lambda g: (0, 0)),
            pl.BlockSpec((1, H), lambda g: (0, 0)),
            pl.BlockSpec((1, H), lambda g: (0, 0)),
            pl.BlockSpec((1, H), lambda g: (0, 0)),
        ],
        out_specs=pl.BlockSpec((NPW, H), lambda g: (g, 0)),
        out_shape=jax.ShapeDtypeStruct((N, H), jnp.float32),
    )(e, pooledt, W1a, W1b, b12, gamma2, beta2, mm2, mv2)


def kernel(embeddings, weights, neighbor_set, WQ, bQ, WK, bK, WV, bV, W1, b1,
           gamma, beta, moving_mean, moving_var):
    e = embeddings[0]
    ns_r = (neighbor_set[0]
            .reshape(NW, NGROUPS, GROUP, NOCT, KO)
            .transpose(0, 1, 3, 2, 4)
            .reshape(NW, NCHUNK, CHUNK))

    eqt_blocks, ek, ev = _project(
        e, WQ, bQ.reshape(H, 1), WK, bK.reshape(1, H), WV, bV.reshape(1, H))
    evek = lax.bitcast_convert_type(
        jnp.stack([ev, ek], axis=-1), jnp.float32)

    pooledt = _sc_attend(ns_r, eqt_blocks, evek)

    out = _post(
        e, pooledt, W1[:D], W1[D:], b1.reshape(1, H),
        gamma.reshape(1, H), beta.reshape(1, H),
        moving_mean.reshape(1, H), moving_var.reshape(1, H))
    return out.reshape(1, N, H)


# speedup vs baseline: 1.2169x; 1.0275x over previous
import functools

import jax
import jax.numpy as jnp
import numpy as np
from jax import lax
from jax.experimental import pallas as pl
from jax.experimental.pallas import tpu as pltpu
from jax.experimental.pallas import tpu_sc as plsc

N = 4096
K = 32
D = 128
H = 128
NW = 32
NPW = N // NW
GROUP = 16
NGROUPS = NPW // GROUP
ROWS = GROUP * K
CHUNK = 128
NCHUNK = NPW * K // CHUNK


def _leaky(x):
    return jnp.where(x >= 0, x, 0.3 * x)


def _project_body(e_ref, wq_ref, bqc_ref, wk_ref, bk_ref, wv_ref, bv_ref,
                  eqt_ref, evek_ref):
    e = e_ref[...]
    ek_b = _leaky(
        jnp.dot(e, wk_ref[...], preferred_element_type=jnp.float32)
        + bk_ref[...]).astype(jnp.bfloat16)
    ev_b = _leaky(
        jnp.dot(e, wv_ref[...], preferred_element_type=jnp.float32)
        + bv_ref[...]).astype(jnp.bfloat16)
    ek_u = lax.convert_element_type(
        lax.bitcast_convert_type(ek_b, jnp.uint16), jnp.uint32)
    ev_u = lax.convert_element_type(
        lax.bitcast_convert_type(ev_b, jnp.uint16), jnp.uint32)
    evek_ref[...] = lax.bitcast_convert_type(
        (ek_u << 16) | ev_u, jnp.float32)
    eqt = lax.dot_general(wq_ref[...], e, (((0,), (1,)), ((), ())),
                          preferred_element_type=jnp.float32)
    eqt_ref[0] = _leaky(eqt + bqc_ref[...])


def _project(e, WQ, bQc, WK, bK2, WV, bV2):
    return pl.pallas_call(
        _project_body,
        grid=(NW,),
        in_specs=[
            pl.BlockSpec((NPW, D), lambda g: (g, 0)),
            pl.BlockSpec((D, H), lambda g: (0, 0)),
            pl.BlockSpec((H, 1), lambda g: (0, 0)),
            pl.BlockSpec((D, H), lambda g: (0, 0)),
            pl.BlockSpec((1, H), lambda g: (0, 0)),
            pl.BlockSpec((D, H), lambda g: (0, 0)),
            pl.BlockSpec((1, H), lambda g: (0, 0)),
        ],
        out_specs=[
            pl.BlockSpec((1, H, NPW), lambda g: (g, 0, 0)),
            pl.BlockSpec((NPW, H), lambda g: (g, 0)),
        ],
        out_shape=[
            jax.ShapeDtypeStruct((NW, H, NPW), jnp.float32),
            jax.ShapeDtypeStruct((N, H), jnp.float32),
        ],
    )(e, WQ, bQc, WK, bK2, WV, bV2)


NOCT = 4
KO = K // NOCT
_MASK_HI = np.uint32(0xFFFF0000)


def _unpack_lo(w):
    u = lax.bitcast_convert_type(w, jnp.uint32)
    return lax.bitcast_convert_type(u << 16, jnp.float32)


def _unpack_hi(w):
    u = lax.bitcast_convert_type(w, jnp.uint32)
    return lax.bitcast_convert_type(u & _MASK_HI, jnp.float32)


def _sc_attend_body(ns_hbm, eqt_hbm, evek_hbm, outt_hbm,
                    idx_v, rows_v, eqt_v, eqtrot_v, ptrot_v, sem):
    wid = lax.axis_index("s") * 2 + lax.axis_index("c")
    pltpu.sync_copy(ns_hbm.at[wid], idx_v)
    pltpu.sync_copy(eqt_hbm.at[wid], eqt_v)

    lanes = lax.iota(jnp.int32, 16)

    def fire(g, c):
        return pltpu.async_copy(
            evek_hbm.at[idx_v.at[NOCT * g + c]],
            rows_v.at[pl.ds(c * CHUNK, CHUNK)],
            sem)

    def rot_h(h, carry):
        hrot = (h + lanes) & (H - 1)
        for s8 in range(NPW // 16):
            col = jnp.full((16,), s8 * 16, jnp.int32) + lanes
            eqtrot_v[h, pl.ds(s8 * 16, 16)] = plsc.load_gather(
                eqt_v, [hrot, col])
        return carry

    lax.fori_loop(0, H, rot_h, 0)

    pending = [fire(0, c) for c in range(NOCT)]

    for g in range(NGROUPS):
        g16 = g * GROUP

        s = []
        for c in range(NOCT):
            pending[c].wait()

            def score_h(i, carry, c=c):
                cr0 = carry[0]
                cr1 = (cr0 + 1) & (H - 1)
                sc = carry[1:]
                eqt0 = eqtrot_v[2 * i, pl.ds(g16, GROUP)]
                eqt1 = eqtrot_v[2 * i + 1, pl.ds(g16, GROUP)]
                out = []
                for kk in range(KO):
                    row = lanes * KO + (c * CHUNK + kk)
                    v0 = eqt0 * _unpack_lo(plsc.load_gather(rows_v, [row, cr0]))
                    v1 = eqt1 * _unpack_lo(plsc.load_gather(rows_v, [row, cr1]))
                    out.append(sc[kk] + (v0 + v1))
                return ((cr1 + 1) & (H - 1),) + tuple(out)

            s0 = (lanes,) + tuple(jnp.zeros((16,), jnp.float32)
                                  for _ in range(KO))
            s.extend(lax.fori_loop(0, H // 2, score_h, s0)[1:])

        m = s[0]
        for k in range(1, K):
            m = jnp.maximum(m, s[k])
        e = [jnp.exp(s[k] - m) for k in range(K)]
        den = e[0]
        for k in range(1, K):
            den = den + e[k]
        inv = 1.0 / den
        a = [e[k] * inv for k in range(K)]

        for c in range(NOCT):
            ac = a[c * KO:(c + 1) * KO]

            def pool_h(i, cr0, c=c, ac=ac):
                cr1 = (cr0 + 1) & (H - 1)
                acc0 = jnp.zeros((16,), jnp.float32)
                acc1 = jnp.zeros((16,), jnp.float32)
                for kk in range(KO):
                    row = lanes * KO + (c * CHUNK + kk)
                    acc0 = acc0 + ac[kk] * _unpack_hi(
                        plsc.load_gather(rows_v, [row, cr0]))
                    acc1 = acc1 + ac[kk] * _unpack_hi(
                        plsc.load_gather(rows_v, [row, cr1]))
                if c == 0:
                    ptrot_v[2 * i, pl.ds(g16, GROUP)] = acc0
                    ptrot_v[2 * i + 1, pl.ds(g16, GROUP)] = acc1
                else:
                    ptrot_v[2 * i, pl.ds(g16, GROUP)] += acc0
                    ptrot_v[2 * i + 1, pl.ds(g16, GROUP)] += acc1
                return (cr1 + 1) & (H - 1)

            lax.fori_loop(0, H // 2, pool_h, lanes)
            if g + 1 < NGROUPS:
                pending[c] = fire(g + 1, c)

    def unrot_h(h, carry):
        hrot = (h - lanes) & (H - 1)
        for s8 in range(NPW // 16):
            col = jnp.full((16,), s8 * 16, jnp.int32) + lanes
            eqt_v[h, pl.ds(s8 * 16, 16)] = plsc.load_gather(
                ptrot_v, [hrot, col])
        return carry

    lax.fori_loop(0, H, unrot_h, 0)

    pltpu.sync_copy(eqt_v, outt_hbm.at[wid])


def _sc_attend(ns_r, eqt_blocks, evek):
    mesh = plsc.VectorSubcoreMesh(core_axis_name="c", subcore_axis_name="s")
    run = functools.partial(
        pl.kernel,
        mesh=mesh,
        compiler_params=pltpu.CompilerParams(needs_layout_passes=False),
        out_type=jax.ShapeDtypeStruct((NW, H, NPW), jnp.float32),
        scratch_types=[
            pltpu.VMEM((NCHUNK, CHUNK), jnp.int32),
            pltpu.VMEM((ROWS, H), jnp.float32),
            pltpu.VMEM((H, NPW), jnp.float32),
            pltpu.VMEM((H, NPW), jnp.float32),
            pltpu.VMEM((H, NPW), jnp.float32),
            pltpu.SemaphoreType.DMA,
        ],
    )(_sc_attend_body)
    return run(ns_r, eqt_blocks, evek)


def _post_body(e_ref, pt_ref, w1a_ref, w1b_ref, b1_ref,
               gamma_ref, beta_ref, mm_ref, mv_ref, out_ref):
    e = e_ref[...]
    hidden = _leaky(
        jnp.dot(e, w1a_ref[...], preferred_element_type=jnp.float32)
        + lax.dot_general(pt_ref[0], w1b_ref[...], (((0,), (0,)), ((), ())),
                          preferred_element_type=jnp.float32)
        + b1_ref[...])
    nrm = jnp.sqrt(jnp.sum(hidden * hidden, axis=1, keepdims=True))
    normalized = hidden / (nrm + 1e-6)
    out_ref[...] = (gamma_ref[...] * (normalized - mm_ref[...])
                    / jnp.sqrt(mv_ref[...] + 1e-3) + beta_ref[...])


def _post(e, pooledt, W1a, W1b, b12, gamma2, beta2, mm2, mv2):
    return pl.pallas_call(
        _post_body,
        grid=(NW,),
        in_specs=[
            pl.BlockSpec((NPW, D), lambda g: (g, 0)),
            pl.BlockSpec((1, H, NPW), lambda g: (g, 0, 0)),
            pl.BlockSpec((D, H), lambda g: (0, 0)),
            pl.BlockSpec((H, H), lambda g: (0, 0)),
            pl.BlockSpec((1, H), lambda g: (0, 0)),
            pl.BlockSpec((1, H), lambda g: (0, 0)),
            pl.BlockSpec((1, H), lambda g: (0, 0)),
            pl.BlockSpec((1, H), lambda g: (0, 0)),
            pl.BlockSpec((1, H), lambda g: (0, 0)),
        ],
        out_specs=pl.BlockSpec((NPW, H), lambda g: (g, 0)),
        out_shape=jax.ShapeDtypeStruct((N, H), jnp.float32),
    )(e, pooledt, W1a, W1b, b12, gamma2, beta2, mm2, mv2)


def kernel(embeddings, weights, neighbor_set, WQ, bQ, WK, bK, WV, bV, W1, b1,
           gamma, beta, moving_mean, moving_var):
    e = embeddings[0]
    ns_r = (neighbor_set[0]
            .reshape(NW, NGROUPS, GROUP, NOCT, KO)
            .transpose(0, 1, 3, 2, 4)
            .reshape(NW, NCHUNK, CHUNK))

    eqt_blocks, evek = _project(
        e, WQ, bQ.reshape(H, 1), WK, bK.reshape(1, H), WV, bV.reshape(1, H))

    pooledt = _sc_attend(ns_r, eqt_blocks, evek)

    out = _post(
        e, pooledt, W1[:D], W1[D:], b1.reshape(1, H),
        gamma.reshape(1, H), beta.reshape(1, H),
        moving_mean.reshape(1, H), moving_var.reshape(1, H))
    return out.reshape(1, N, H)


# speedup vs baseline: 1.2184x; 1.0012x over previous
import functools

import jax
import jax.numpy as jnp
import numpy as np
from jax import lax
from jax.experimental import pallas as pl
from jax.experimental.pallas import tpu as pltpu
from jax.experimental.pallas import tpu_sc as plsc

N = 4096
K = 32
D = 128
H = 128
NW = 32
NPW = N // NW
GROUP = 16
NGROUPS = NPW // GROUP
ROWS = GROUP * K
CHUNK = 128
NCHUNK = NPW * K // CHUNK


def _leaky(x):
    return jnp.where(x >= 0, x, 0.3 * x)


def _project_body(e_ref, wq_ref, bqc_ref, wk_ref, bk_ref, wv_ref, bv_ref,
                  eqt_ref, evek_ref):
    e = e_ref[...]
    ek_b = _leaky(
        jnp.dot(e, wk_ref[...], preferred_element_type=jnp.float32)
        + bk_ref[...]).astype(jnp.bfloat16)
    ev_b = _leaky(
        jnp.dot(e, wv_ref[...], preferred_element_type=jnp.float32)
        + bv_ref[...]).astype(jnp.bfloat16)
    ek_u = lax.convert_element_type(
        lax.bitcast_convert_type(ek_b, jnp.uint16), jnp.uint32)
    ev_u = lax.convert_element_type(
        lax.bitcast_convert_type(ev_b, jnp.uint16), jnp.uint32)
    evek_ref[...] = lax.bitcast_convert_type(
        (ek_u << 16) | ev_u, jnp.float32)
    eqt = lax.dot_general(wq_ref[...], e, (((0,), (1,)), ((), ())),
                          preferred_element_type=jnp.float32)
    eqt_ref[0] = _leaky(eqt + bqc_ref[...])


def _project(e, WQ, bQc, WK, bK2, WV, bV2):
    return pl.pallas_call(
        _project_body,
        grid=(NW,),
        in_specs=[
            pl.BlockSpec((NPW, D), lambda g: (g, 0)),
            pl.BlockSpec((D, H), lambda g: (0, 0)),
            pl.BlockSpec((H, 1), lambda g: (0, 0)),
            pl.BlockSpec((D, H), lambda g: (0, 0)),
            pl.BlockSpec((1, H), lambda g: (0, 0)),
            pl.BlockSpec((D, H), lambda g: (0, 0)),
            pl.BlockSpec((1, H), lambda g: (0, 0)),
        ],
        out_specs=[
            pl.BlockSpec((1, H, NPW), lambda g: (g, 0, 0)),
            pl.BlockSpec((NPW, H), lambda g: (g, 0)),
        ],
        out_shape=[
            jax.ShapeDtypeStruct((NW, H, NPW), jnp.float32),
            jax.ShapeDtypeStruct((N, H), jnp.float32),
        ],
    )(e, WQ, bQc, WK, bK2, WV, bV2)


NOCT = 4
KO = K // NOCT
_MASK_HI = np.uint32(0xFFFF0000)


def _unpack_lo(w):
    u = lax.bitcast_convert_type(w, jnp.uint32)
    return lax.bitcast_convert_type(u << 16, jnp.float32)


def _unpack_hi(w):
    u = lax.bitcast_convert_type(w, jnp.uint32)
    return lax.bitcast_convert_type(u & _MASK_HI, jnp.float32)


def _sc_attend_body(ns_hbm, eqt_hbm, evek_hbm, outt_hbm,
                    idx_v, rows_v, eqt_v, eqtrot_v, ptrot_v, sem):
    wid = lax.axis_index("s") * 2 + lax.axis_index("c")
    pltpu.sync_copy(ns_hbm.at[wid], idx_v)
    pltpu.sync_copy(eqt_hbm.at[wid], eqt_v)

    lanes = lax.iota(jnp.int32, 16)

    def fire(g, c):
        return pltpu.async_copy(
            evek_hbm.at[idx_v.at[NOCT * g + c]],
            rows_v.at[pl.ds(c * CHUNK, CHUNK)],
            sem)

    def rot_h(h, carry):
        hrot = (h + lanes) & (H - 1)
        for s8 in range(NPW // 16):
            col = jnp.full((16,), s8 * 16, jnp.int32) + lanes
            eqtrot_v[h, pl.ds(s8 * 16, 16)] = plsc.load_gather(
                eqt_v, [hrot, col])
        return carry

    lax.fori_loop(0, H, rot_h, 0)

    pending = [fire(0, c) for c in range(NOCT)]

    for g in range(NGROUPS):
        g16 = g * GROUP

        s = []
        for c in range(NOCT):
            pending[c].wait()

            def score_h(i, carry, c=c):
                cr = [carry[0]]
                for _ in range(3):
                    cr.append((cr[-1] + 1) & (H - 1))
                sc = carry[1:]
                eqt = [eqtrot_v[4 * i + j, pl.ds(g16, GROUP)]
                       for j in range(4)]
                out = []
                for kk in range(KO):
                    row = lanes * KO + (c * CHUNK + kk)
                    v = sc[kk]
                    for j in range(4):
                        v = v + eqt[j] * _unpack_lo(
                            plsc.load_gather(rows_v, [row, cr[j]]))
                    out.append(v)
                return ((cr[3] + 1) & (H - 1),) + tuple(out)

            s0 = (lanes,) + tuple(jnp.zeros((16,), jnp.float32)
                                  for _ in range(KO))
            s.extend(lax.fori_loop(0, H // 4, score_h, s0)[1:])

        m = s[0]
        for k in range(1, K):
            m = jnp.maximum(m, s[k])
        e = [jnp.exp(s[k] - m) for k in range(K)]
        den = e[0]
        for k in range(1, K):
            den = den + e[k]
        inv = 1.0 / den
        a = [e[k] * inv for k in range(K)]

        for c in range(NOCT):
            ac = a[c * KO:(c + 1) * KO]

            def pool_h(i, cr0, c=c, ac=ac):
                cr = [cr0]
                for _ in range(3):
                    cr.append((cr[-1] + 1) & (H - 1))
                acc = [jnp.zeros((16,), jnp.float32) for _ in range(4)]
                for kk in range(KO):
                    row = lanes * KO + (c * CHUNK + kk)
                    for j in range(4):
                        acc[j] = acc[j] + ac[kk] * _unpack_hi(
                            plsc.load_gather(rows_v, [row, cr[j]]))
                for j in range(4):
                    if c == 0:
                        ptrot_v[4 * i + j, pl.ds(g16, GROUP)] = acc[j]
                    else:
                        ptrot_v[4 * i + j, pl.ds(g16, GROUP)] += acc[j]
                return (cr[3] + 1) & (H - 1)

            lax.fori_loop(0, H // 4, pool_h, lanes)
            if g + 1 < NGROUPS:
                pending[c] = fire(g + 1, c)

    def unrot_h(h, carry):
        hrot = (h - lanes) & (H - 1)
        for s8 in range(NPW // 16):
            col = jnp.full((16,), s8 * 16, jnp.int32) + lanes
            eqt_v[h, pl.ds(s8 * 16, 16)] = plsc.load_gather(
                ptrot_v, [hrot, col])
        return carry

    lax.fori_loop(0, H, unrot_h, 0)

    pltpu.sync_copy(eqt_v, outt_hbm.at[wid])


def _sc_attend(ns_r, eqt_blocks, evek):
    mesh = plsc.VectorSubcoreMesh(core_axis_name="c", subcore_axis_name="s")
    run = functools.partial(
        pl.kernel,
        mesh=mesh,
        compiler_params=pltpu.CompilerParams(needs_layout_passes=False),
        out_type=jax.ShapeDtypeStruct((NW, H, NPW), jnp.float32),
        scratch_types=[
            pltpu.VMEM((NCHUNK, CHUNK), jnp.int32),
            pltpu.VMEM((ROWS, H), jnp.float32),
            pltpu.VMEM((H, NPW), jnp.float32),
            pltpu.VMEM((H, NPW), jnp.float32),
            pltpu.VMEM((H, NPW), jnp.float32),
            pltpu.SemaphoreType.DMA,
        ],
    )(_sc_attend_body)
    return run(ns_r, eqt_blocks, evek)


def _post_body(e_ref, pt_ref, w1a_ref, w1b_ref, b1_ref,
               gamma_ref, beta_ref, mm_ref, mv_ref, out_ref):
    e = e_ref[...]
    hidden = _leaky(
        jnp.dot(e, w1a_ref[...], preferred_element_type=jnp.float32)
        + lax.dot_general(pt_ref[0], w1b_ref[...], (((0,), (0,)), ((), ())),
                          preferred_element_type=jnp.float32)
        + b1_ref[...])
    nrm = jnp.sqrt(jnp.sum(hidden * hidden, axis=1, keepdims=True))
    normalized = hidden / (nrm + 1e-6)
    out_ref[...] = (gamma_ref[...] * (normalized - mm_ref[...])
                    / jnp.sqrt(mv_ref[...] + 1e-3) + beta_ref[...])


def _post(e, pooledt, W1a, W1b, b12, gamma2, beta2, mm2, mv2):
    return pl.pallas_call(
        _post_body,
        grid=(NW,),
        in_specs=[
            pl.BlockSpec((NPW, D), lambda g: (g, 0)),
            pl.BlockSpec((1, H, NPW), lambda g: (g, 0, 0)),
            pl.BlockSpec((D, H), lambda g: (0, 0)),
            pl.BlockSpec((H, H), lambda g: (0, 0)),
            pl.BlockSpec((1, H), lambda g: (0, 0)),
            pl.BlockSpec((1, H), lambda g: (0, 0)),
            pl.BlockSpec((1, H), lambda g: (0, 0)),
            pl.BlockSpec((1, H), lambda g: (0, 0)),
            pl.BlockSpec((1, H), lambda g: (0, 0)),
        ],
        out_specs=pl.BlockSpec((NPW, H), lambda g: (g, 0)),
        out_shape=jax.ShapeDtypeStruct((N, H), jnp.float32),
    )(e, pooledt, W1a, W1b, b12, gamma2, beta2, mm2, mv2)


def kernel(embeddings, weights, neighbor_set, WQ, bQ, WK, bK, WV, bV, W1, b1,
           gamma, beta, moving_mean, moving_var):
    e = embeddings[0]
    ns_r = (neighbor_set[0]
            .reshape(NW, NGROUPS, GROUP, NOCT, KO)
            .transpose(0, 1, 3, 2, 4)
            .reshape(NW, NCHUNK, CHUNK))

    eqt_blocks, evek = _project(
        e, WQ, bQ.reshape(H, 1), WK, bK.reshape(1, H), WV, bV.reshape(1, H))

    pooledt = _sc_attend(ns_r, eqt_blocks, evek)

    out = _post(
        e, pooledt, W1[:D], W1[D:], b1.reshape(1, H),
        gamma.reshape(1, H), beta.reshape(1, H),
        moving_mean.reshape(1, H), moving_var.reshape(1, H))
    return out.reshape(1, N, H)


# speedup vs baseline: 1.5657x; 1.2851x over previous
import functools

import jax
import jax.numpy as jnp
import numpy as np
from jax import lax
from jax.experimental import pallas as pl
from jax.experimental.pallas import tpu as pltpu
from jax.experimental.pallas import tpu_sc as plsc

N = 4096
K = 32
D = 128
H = 128
NW = 32
NPW = N // NW
GROUP = 16
NGROUPS = NPW // GROUP
ROWS = GROUP * K
CHUNK = 128
NCHUNK = NPW * K // CHUNK


def _leaky(x):
    return jnp.where(x >= 0, x, 0.3 * x)


def _project_body(e_ref, wq_ref, bq_ref, wk_ref, bk_ref, wv_ref, bv_ref,
                  eq_ref, evek_ref):
    e = e_ref[...]
    eq_ref[...] = _leaky(
        jnp.dot(e, wq_ref[...], preferred_element_type=jnp.float32)
        + bq_ref[...])
    ek_b = _leaky(
        jnp.dot(e, wk_ref[...], preferred_element_type=jnp.float32)
        + bk_ref[...]).astype(jnp.bfloat16)
    ev_b = _leaky(
        jnp.dot(e, wv_ref[...], preferred_element_type=jnp.float32)
        + bv_ref[...]).astype(jnp.bfloat16)
    ek_u = lax.convert_element_type(
        lax.bitcast_convert_type(ek_b, jnp.uint16), jnp.uint32)
    ev_u = lax.convert_element_type(
        lax.bitcast_convert_type(ev_b, jnp.uint16), jnp.uint32)
    evek_ref[...] = lax.bitcast_convert_type(
        (ek_u << 16) | ev_u, jnp.float32)


def _project(e, WQ, bQ2, WK, bK2, WV, bV2):
    return pl.pallas_call(
        _project_body,
        out_shape=[
            jax.ShapeDtypeStruct((N, H), jnp.float32),
            jax.ShapeDtypeStruct((N, H), jnp.float32),
        ],
    )(e, WQ, bQ2, WK, bK2, WV, bV2)


NOCT = 4
KO = K // NOCT
_MASK_HI = np.uint32(0xFFFF0000)


def _unpack_lo(w):
    u = lax.bitcast_convert_type(w, jnp.uint32)
    return lax.bitcast_convert_type(u << 16, jnp.float32)


def _unpack_hi(w):
    u = lax.bitcast_convert_type(w, jnp.uint32)
    return lax.bitcast_convert_type(u & _MASK_HI, jnp.float32)


def _sc_attend_body(ns_hbm, eq_hbm, evek_hbm, outt_hbm,
                    idx_v, rows_v, eq_v, eqtrot_v, ptrot_v, sem):
    wid = lax.axis_index("s") * 2 + lax.axis_index("c")
    pltpu.sync_copy(ns_hbm.at[wid], idx_v)
    pltpu.sync_copy(eq_hbm.at[pl.ds(wid * NPW, NPW)], eq_v)

    lanes = lax.iota(jnp.int32, 16)

    def fire(g, c):
        return pltpu.async_copy(
            evek_hbm.at[idx_v.at[NOCT * g + c]],
            rows_v.at[pl.ds(c * CHUNK, CHUNK)],
            sem)

    def rot_h(h, carry):
        hrot = (h + lanes) & (H - 1)
        for s8 in range(NPW // 16):
            node = jnp.full((16,), s8 * 16, jnp.int32) + lanes
            eqtrot_v[h, pl.ds(s8 * 16, 16)] = plsc.load_gather(
                eq_v, [node, hrot])
        return carry

    lax.fori_loop(0, H, rot_h, 0)

    pending = [fire(0, c) for c in range(NOCT)]

    for g in range(NGROUPS):
        g16 = g * GROUP

        s = []
        for c in range(NOCT):
            pending[c].wait()

            def score_h(i, carry, c=c):
                cr = [carry[0]]
                for _ in range(3):
                    cr.append((cr[-1] + 1) & (H - 1))
                sc = carry[1:]
                eqt = [eqtrot_v[4 * i + j, pl.ds(g16, GROUP)]
                       for j in range(4)]
                out = []
                for kk in range(KO):
                    row = lanes * KO + (c * CHUNK + kk)
                    v = sc[kk]
                    for j in range(4):
                        v = v + eqt[j] * _unpack_lo(
                            plsc.load_gather(rows_v, [row, cr[j]]))
                    out.append(v)
                return ((cr[3] + 1) & (H - 1),) + tuple(out)

            s0 = (lanes,) + tuple(jnp.zeros((16,), jnp.float32)
                                  for _ in range(KO))
            s.extend(lax.fori_loop(0, H // 4, score_h, s0)[1:])

        m = s[0]
        for k in range(1, K):
            m = jnp.maximum(m, s[k])
        e = [jnp.exp(s[k] - m) for k in range(K)]
        den = e[0]
        for k in range(1, K):
            den = den + e[k]
        inv = 1.0 / den
        a = [e[k] * inv for k in range(K)]

        for c in range(NOCT):
            ac = a[c * KO:(c + 1) * KO]

            def pool_h(i, cr0, c=c, ac=ac):
                cr = [cr0]
                for _ in range(3):
                    cr.append((cr[-1] + 1) & (H - 1))
                acc = [jnp.zeros((16,), jnp.float32) for _ in range(4)]
                for kk in range(KO):
                    row = lanes * KO + (c * CHUNK + kk)
                    for j in range(4):
                        acc[j] = acc[j] + ac[kk] * _unpack_hi(
                            plsc.load_gather(rows_v, [row, cr[j]]))
                for j in range(4):
                    if c == 0:
                        ptrot_v[4 * i + j, pl.ds(g16, GROUP)] = acc[j]
                    else:
                        ptrot_v[4 * i + j, pl.ds(g16, GROUP)] += acc[j]
                return (cr[3] + 1) & (H - 1)

            lax.fori_loop(0, H // 4, pool_h, lanes)
            if g + 1 < NGROUPS:
                pending[c] = fire(g + 1, c)

    def unrot_h(h, carry):
        hrot = (h - lanes) & (H - 1)
        for s8 in range(NPW // 16):
            col = jnp.full((16,), s8 * 16, jnp.int32) + lanes
            eq_v[h, pl.ds(s8 * 16, 16)] = plsc.load_gather(
                ptrot_v, [hrot, col])
        return carry

    lax.fori_loop(0, H, unrot_h, 0)

    pltpu.sync_copy(eq_v, outt_hbm.at[wid])


def _sc_attend(ns_r, eq, evek):
    mesh = plsc.VectorSubcoreMesh(core_axis_name="c", subcore_axis_name="s")
    run = functools.partial(
        pl.kernel,
        mesh=mesh,
        compiler_params=pltpu.CompilerParams(needs_layout_passes=False),
        out_type=jax.ShapeDtypeStruct((NW, H, NPW), jnp.float32),
        scratch_types=[
            pltpu.VMEM((NCHUNK, CHUNK), jnp.int32),
            pltpu.VMEM((ROWS, H), jnp.float32),
            pltpu.VMEM((H, NPW), jnp.float32),
            pltpu.VMEM((H, NPW), jnp.float32),
            pltpu.VMEM((H, NPW), jnp.float32),
            pltpu.SemaphoreType.DMA,
        ],
    )(_sc_attend_body)
    return run(ns_r, eq, evek)


def _post_body(e_ref, pt_ref, w1a_ref, w1b_ref, b1_ref,
               gamma_ref, beta_ref, mm_ref, mv_ref, out_ref):
    e = e_ref[...]
    pooled = lax.dot_general(
        pt_ref[...], w1b_ref[...], (((1,), (0,)), ((), ())),
        preferred_element_type=jnp.float32).reshape(N, H)
    hidden = _leaky(
        jnp.dot(e, w1a_ref[...], preferred_element_type=jnp.float32)
        + pooled + b1_ref[...])
    nrm = jnp.sqrt(jnp.sum(hidden * hidden, axis=1, keepdims=True))
    normalized = hidden / (nrm + 1e-6)
    out_ref[...] = (gamma_ref[...] * (normalized - mm_ref[...])
                    / jnp.sqrt(mv_ref[...] + 1e-3) + beta_ref[...])


def _post(e, pooledt, W1a, W1b, b12, gamma2, beta2, mm2, mv2):
    return pl.pallas_call(
        _post_body,
        out_shape=jax.ShapeDtypeStruct((N, H), jnp.float32),
    )(e, pooledt, W1a, W1b, b12, gamma2, beta2, mm2, mv2)


def kernel(embeddings, weights, neighbor_set, WQ, bQ, WK, bK, WV, bV, W1, b1,
           gamma, beta, moving_mean, moving_var):
    e = embeddings[0]
    ns_r = (neighbor_set[0]
            .reshape(NW, NGROUPS, GROUP, NOCT, KO)
            .transpose(0, 1, 3, 2, 4)
            .reshape(NW, NCHUNK, CHUNK))

    eq, evek = _project(
        e, WQ, bQ.reshape(1, H), WK, bK.reshape(1, H), WV, bV.reshape(1, H))

    pooledt = _sc_attend(ns_r, eq, evek)

    out = _post(
        e, pooledt, W1[:D], W1[D:], b1.reshape(1, H),
        gamma.reshape(1, H), beta.reshape(1, H),
        moving_mean.reshape(1, H), moving_var.reshape(1, H))
    return out.reshape(1, N, H)


# speedup vs baseline: 1.5927x; 1.0172x over previous
import functools

import jax
import jax.numpy as jnp
import numpy as np
from jax import lax
from jax.experimental import pallas as pl
from jax.experimental.pallas import tpu as pltpu
from jax.experimental.pallas import tpu_sc as plsc

N = 4096
K = 32
D = 128
H = 128
NW = 32
NPW = N // NW
GROUP = 16
NGROUPS = NPW // GROUP
ROWS = GROUP * K
CHUNK = 128
NCHUNK = NPW * K // CHUNK


def _leaky(x):
    return jnp.where(x >= 0, x, 0.3 * x)


def _project_body(e_ref, wq_ref, bq_ref, wk_ref, bk_ref, wv_ref, bv_ref,
                  eq_ref, evek_ref):
    e = e_ref[...]
    eq_ref[...] = _leaky(
        jnp.dot(e, wq_ref[...], preferred_element_type=jnp.float32)
        + bq_ref[...])
    ek_b = _leaky(
        jnp.dot(e, wk_ref[...], preferred_element_type=jnp.float32)
        + bk_ref[...]).astype(jnp.bfloat16)
    ev_b = _leaky(
        jnp.dot(e, wv_ref[...], preferred_element_type=jnp.float32)
        + bv_ref[...]).astype(jnp.bfloat16)
    ek_u = lax.convert_element_type(
        lax.bitcast_convert_type(ek_b, jnp.uint16), jnp.uint32)
    ev_u = lax.convert_element_type(
        lax.bitcast_convert_type(ev_b, jnp.uint16), jnp.uint32)
    evek_ref[...] = lax.bitcast_convert_type(
        (ek_u << 16) | ev_u, jnp.float32)


def _project(e, WQ, bQ2, WK, bK2, WV, bV2):
    return pl.pallas_call(
        _project_body,
        out_shape=[
            jax.ShapeDtypeStruct((N, H), jnp.float32),
            jax.ShapeDtypeStruct((N, H), jnp.float32),
        ],
    )(e, WQ, bQ2, WK, bK2, WV, bV2)


NOCT = 4
KO = K // NOCT
_MASK_HI = np.uint32(0xFFFF0000)


def _unpack_lo(w):
    u = lax.bitcast_convert_type(w, jnp.uint32)
    return lax.bitcast_convert_type(u << 16, jnp.float32)


def _unpack_hi(w):
    u = lax.bitcast_convert_type(w, jnp.uint32)
    return lax.bitcast_convert_type(u & _MASK_HI, jnp.float32)


def _sc_attend_body(ns_hbm, eq_hbm, evek_hbm, outt_hbm,
                    idx_v, rows_v, eq_v, eqtrot_v, ptrot_v, sem):
    wid = lax.axis_index("s") * 2 + lax.axis_index("c")
    pltpu.sync_copy(ns_hbm.at[wid], idx_v)
    pltpu.sync_copy(eq_hbm.at[pl.ds(wid * NPW, NPW)], eq_v)

    lanes = lax.iota(jnp.int32, 16)

    def fire(g, c):
        return pltpu.async_copy(
            evek_hbm.at[idx_v.at[NOCT * g + c]],
            rows_v.at[pl.ds(c * CHUNK, CHUNK)],
            sem)

    def rot_h(h, carry):
        hrot = (h + lanes) & (H - 1)
        for s8 in range(NPW // 16):
            node = jnp.full((16,), s8 * 16, jnp.int32) + lanes
            eqtrot_v[h, pl.ds(s8 * 16, 16)] = plsc.load_gather(
                eq_v, [node, hrot])
        return carry

    pending = [fire(0, c) for c in range(NOCT)]

    lax.fori_loop(0, H, rot_h, 0)

    for g in range(NGROUPS):
        g16 = g * GROUP

        s = []
        for c in range(NOCT):
            pending[c].wait()

            def score_h(i, carry, c=c):
                cr = [carry[0]]
                for _ in range(3):
                    cr.append((cr[-1] + 1) & (H - 1))
                sc = carry[1:]
                eqt = [eqtrot_v[4 * i + j, pl.ds(g16, GROUP)]
                       for j in range(4)]
                out = []
                for kk in range(KO):
                    row = lanes * KO + (c * CHUNK + kk)
                    v = sc[kk]
                    for j in range(4):
                        v = v + eqt[j] * _unpack_lo(
                            plsc.load_gather(rows_v, [row, cr[j]]))
                    out.append(v)
                return ((cr[3] + 1) & (H - 1),) + tuple(out)

            s0 = (lanes,) + tuple(jnp.zeros((16,), jnp.float32)
                                  for _ in range(KO))
            s.extend(lax.fori_loop(0, H // 4, score_h, s0)[1:])

        m = s[0]
        for k in range(1, K):
            m = jnp.maximum(m, s[k])
        e = [jnp.exp(s[k] - m) for k in range(K)]
        den = e[0]
        for k in range(1, K):
            den = den + e[k]
        inv = 1.0 / den
        a = [e[k] * inv for k in range(K)]

        for c in range(NOCT):
            ac = a[c * KO:(c + 1) * KO]

            def pool_h(i, cr0, c=c, ac=ac):
                cr = [cr0]
                for _ in range(3):
                    cr.append((cr[-1] + 1) & (H - 1))
                acc = [jnp.zeros((16,), jnp.float32) for _ in range(4)]
                for kk in range(KO):
                    row = lanes * KO + (c * CHUNK + kk)
                    for j in range(4):
                        acc[j] = acc[j] + ac[kk] * _unpack_hi(
                            plsc.load_gather(rows_v, [row, cr[j]]))
                for j in range(4):
                    if c == 0:
                        ptrot_v[4 * i + j, pl.ds(g16, GROUP)] = acc[j]
                    else:
                        ptrot_v[4 * i + j, pl.ds(g16, GROUP)] += acc[j]
                return (cr[3] + 1) & (H - 1)

            lax.fori_loop(0, H // 4, pool_h, lanes)
            if g + 1 < NGROUPS:
                pending[c] = fire(g + 1, c)

    def unrot_h(h, carry):
        hrot = (h - lanes) & (H - 1)
        for s8 in range(NPW // 16):
            col = jnp.full((16,), s8 * 16, jnp.int32) + lanes
            eq_v[h, pl.ds(s8 * 16, 16)] = plsc.load_gather(
                ptrot_v, [hrot, col])
        return carry

    lax.fori_loop(0, H, unrot_h, 0)

    pltpu.sync_copy(eq_v, outt_hbm.at[wid])


def _sc_attend(ns_r, eq, evek):
    mesh = plsc.VectorSubcoreMesh(core_axis_name="c", subcore_axis_name="s")
    run = functools.partial(
        pl.kernel,
        mesh=mesh,
        compiler_params=pltpu.CompilerParams(needs_layout_passes=False),
        out_type=jax.ShapeDtypeStruct((NW, H, NPW), jnp.float32),
        scratch_types=[
            pltpu.VMEM((NCHUNK, CHUNK), jnp.int32),
            pltpu.VMEM((ROWS, H), jnp.float32),
            pltpu.VMEM((H, NPW), jnp.float32),
            pltpu.VMEM((H, NPW), jnp.float32),
            pltpu.VMEM((H, NPW), jnp.float32),
            pltpu.SemaphoreType.DMA,
        ],
    )(_sc_attend_body)
    return run(ns_r, eq, evek)


def _post_body(e_ref, pt_ref, w1a_ref, w1b_ref, b1_ref,
               gamma_ref, beta_ref, mm_ref, mv_ref, out_ref):
    e = e_ref[...]
    pooled = lax.dot_general(
        pt_ref[...], w1b_ref[...], (((1,), (0,)), ((), ())),
        preferred_element_type=jnp.float32).reshape(N, H)
    hidden = _leaky(
        jnp.dot(e, w1a_ref[...], preferred_element_type=jnp.float32)
        + pooled + b1_ref[...])
    nrm = jnp.sqrt(jnp.sum(hidden * hidden, axis=1, keepdims=True))
    normalized = hidden / (nrm + 1e-6)
    out_ref[...] = (gamma_ref[...] * (normalized - mm_ref[...])
                    / jnp.sqrt(mv_ref[...] + 1e-3) + beta_ref[...])


def _post(e, pooledt, W1a, W1b, b12, gamma2, beta2, mm2, mv2):
    return pl.pallas_call(
        _post_body,
        out_shape=jax.ShapeDtypeStruct((N, H), jnp.float32),
    )(e, pooledt, W1a, W1b, b12, gamma2, beta2, mm2, mv2)


def kernel(embeddings, weights, neighbor_set, WQ, bQ, WK, bK, WV, bV, W1, b1,
           gamma, beta, moving_mean, moving_var):
    e = embeddings[0]
    ns_r = (neighbor_set[0]
            .reshape(NW, NGROUPS, GROUP, NOCT, KO)
            .transpose(0, 1, 3, 2, 4)
            .reshape(NW, NCHUNK, CHUNK))

    eq, evek = _project(
        e, WQ, bQ.reshape(1, H), WK, bK.reshape(1, H), WV, bV.reshape(1, H))

    pooledt = _sc_attend(ns_r, eq, evek)

    out = _post(
        e, pooledt, W1[:D], W1[D:], b1.reshape(1, H),
        gamma.reshape(1, H), beta.reshape(1, H),
        moving_mean.reshape(1, H), moving_var.reshape(1, H))
    return out.reshape(1, N, H)


# speedup vs baseline: 1.6209x; 1.0177x over previous
import functools

import jax
import jax.numpy as jnp
import numpy as np
from jax import lax
from jax.experimental import pallas as pl
from jax.experimental.pallas import tpu as pltpu
from jax.experimental.pallas import tpu_sc as plsc

N = 4096
K = 32
D = 128
H = 128
NW = 32
NPW = N // NW
GROUP = 16
NGROUPS = NPW // GROUP
ROWS = GROUP * K
CHUNK = 128
NCHUNK = NPW * K // CHUNK


def _leaky(x):
    return jnp.where(x >= 0, x, 0.3 * x)


def _project_body(e_ref, wq_ref, bq_ref, wk_ref, bk_ref, wv_ref, bv_ref,
                  eq_ref, evek_ref):
    e = e_ref[0]
    eq_ref[...] = _leaky(
        jnp.dot(e, wq_ref[...], preferred_element_type=jnp.float32)
        + bq_ref[...])
    ek_b = _leaky(
        jnp.dot(e, wk_ref[...], preferred_element_type=jnp.float32)
        + bk_ref[...]).astype(jnp.bfloat16)
    ev_b = _leaky(
        jnp.dot(e, wv_ref[...], preferred_element_type=jnp.float32)
        + bv_ref[...]).astype(jnp.bfloat16)
    ek_u = lax.convert_element_type(
        lax.bitcast_convert_type(ek_b, jnp.uint16), jnp.uint32)
    ev_u = lax.convert_element_type(
        lax.bitcast_convert_type(ev_b, jnp.uint16), jnp.uint32)
    evek_ref[...] = lax.bitcast_convert_type(
        (ek_u << 16) | ev_u, jnp.float32)


def _project(e, WQ, bQ2, WK, bK2, WV, bV2):
    return pl.pallas_call(
        _project_body,
        out_shape=[
            jax.ShapeDtypeStruct((N, H), jnp.float32),
            jax.ShapeDtypeStruct((N, H), jnp.float32),
        ],
    )(e, WQ, bQ2, WK, bK2, WV, bV2)


NOCT = 4
KO = K // NOCT
_MASK_HI = np.uint32(0xFFFF0000)


def _unpack_lo(w):
    u = lax.bitcast_convert_type(w, jnp.uint32)
    return lax.bitcast_convert_type(u << 16, jnp.float32)


def _unpack_hi(w):
    u = lax.bitcast_convert_type(w, jnp.uint32)
    return lax.bitcast_convert_type(u & _MASK_HI, jnp.float32)


def _sc_attend_body(ns_hbm, eq_hbm, evek_hbm, outt_hbm,
                    idx_v, ns_v, rows_v, eqtrot_v, ptrot_v, sem, sem0):
    wid = lax.axis_index("s") * 2 + lax.axis_index("c")
    pltpu.sync_copy(ns_hbm.at[pl.ds(wid * NPW, NPW)], ns_v)
    pltpu.sync_copy(eq_hbm.at[pl.ds(wid * NPW, NPW)],
                    rows_v.at[pl.ds(0, NPW)])

    lanes = lax.iota(jnp.int32, 16)

    def perm_n(n, carry):
        g = n // GROUP
        l = n % GROUP
        col = l * KO + (lanes & (KO - 1))
        for half in range(2):
            row = g * NOCT + 2 * half + lax.shift_right_logical(lanes, 3)
            vals = ns_v[n, pl.ds(half * 16, 16)]
            plsc.store_scatter(idx_v, [row, col], vals)
        return carry

    lax.fori_loop(0, NPW, perm_n, 0)

    def fire(g, c, s=None):
        return pltpu.async_copy(
            evek_hbm.at[idx_v.at[NOCT * g + c]],
            rows_v.at[pl.ds(c * CHUNK, CHUNK)],
            sem if s is None else s)

    def rot_h(h, carry):
        hrot = (h + lanes) & (H - 1)
        for s8 in range(NPW // 16):
            node = jnp.full((16,), s8 * 16, jnp.int32) + lanes
            eqtrot_v[h, pl.ds(s8 * 16, 16)] = plsc.load_gather(
                rows_v, [node, hrot])
        return carry

    pending = [None] + [fire(0, c) for c in range(1, NOCT)]

    lax.fori_loop(0, H, rot_h, 0)
    pending[0] = fire(0, 0, sem0)

    for g in range(NGROUPS):
        g16 = g * GROUP

        s = []
        for c in range(NOCT):
            pending[c].wait()

            def score_h(i, carry, c=c):
                cr = [carry[0]]
                for _ in range(3):
                    cr.append((cr[-1] + 1) & (H - 1))
                sc = carry[1:]
                eqt = [eqtrot_v[4 * i + j, pl.ds(g16, GROUP)]
                       for j in range(4)]
                out = []
                for kk in range(KO):
                    row = lanes * KO + (c * CHUNK + kk)
                    v = sc[kk]
                    for j in range(4):
                        v = v + eqt[j] * _unpack_lo(
                            plsc.load_gather(rows_v, [row, cr[j]]))
                    out.append(v)
                return ((cr[3] + 1) & (H - 1),) + tuple(out)

            s0 = (lanes,) + tuple(jnp.zeros((16,), jnp.float32)
                                  for _ in range(KO))
            s.extend(lax.fori_loop(0, H // 4, score_h, s0)[1:])

        m = s[0]
        for k in range(1, K):
            m = jnp.maximum(m, s[k])
        e = [jnp.exp(s[k] - m) for k in range(K)]
        den = e[0]
        for k in range(1, K):
            den = den + e[k]
        inv = 1.0 / den
        a = [e[k] * inv for k in range(K)]

        for c in range(NOCT):
            ac = a[c * KO:(c + 1) * KO]

            def pool_h(i, cr0, c=c, ac=ac):
                cr = [cr0]
                for _ in range(3):
                    cr.append((cr[-1] + 1) & (H - 1))
                acc = [jnp.zeros((16,), jnp.float32) for _ in range(4)]
                for kk in range(KO):
                    row = lanes * KO + (c * CHUNK + kk)
                    for j in range(4):
                        acc[j] = acc[j] + ac[kk] * _unpack_hi(
                            plsc.load_gather(rows_v, [row, cr[j]]))
                for j in range(4):
                    if c == 0:
                        ptrot_v[4 * i + j, pl.ds(g16, GROUP)] = acc[j]
                    else:
                        ptrot_v[4 * i + j, pl.ds(g16, GROUP)] += acc[j]
                return (cr[3] + 1) & (H - 1)

            lax.fori_loop(0, H // 4, pool_h, lanes)
            if g + 1 < NGROUPS:
                pending[c] = fire(g + 1, c)

    def unrot_h(h, carry):
        hrot = (h - lanes) & (H - 1)
        for s8 in range(NPW // 16):
            col = jnp.full((16,), s8 * 16, jnp.int32) + lanes
            rows_v[h, pl.ds(s8 * 16, 16)] = plsc.load_gather(
                ptrot_v, [hrot, col])
        return carry

    lax.fori_loop(0, H, unrot_h, 0)

    pltpu.sync_copy(rows_v.at[pl.ds(0, NPW)], outt_hbm.at[wid])


def _sc_attend(ns_r, eq, evek):
    mesh = plsc.VectorSubcoreMesh(core_axis_name="c", subcore_axis_name="s")
    run = functools.partial(
        pl.kernel,
        mesh=mesh,
        compiler_params=pltpu.CompilerParams(needs_layout_passes=False),
        out_type=jax.ShapeDtypeStruct((NW, H, NPW), jnp.float32),
        scratch_types=[
            pltpu.VMEM((NCHUNK, CHUNK), jnp.int32),
            pltpu.VMEM((NPW, K), jnp.int32),
            pltpu.VMEM((ROWS, H), jnp.float32),
            pltpu.VMEM((H, NPW), jnp.float32),
            pltpu.VMEM((H, NPW), jnp.float32),
            pltpu.SemaphoreType.DMA,
            pltpu.SemaphoreType.DMA,
        ],
    )(_sc_attend_body)
    return run(ns_r, eq, evek)


def _post_body(e_ref, pt_ref, w1a_ref, w1b_ref, b1_ref,
               gamma_ref, beta_ref, mm_ref, mv_ref, out_ref):
    e = e_ref[0]
    pooled = lax.dot_general(
        pt_ref[...], w1b_ref[...], (((1,), (0,)), ((), ())),
        preferred_element_type=jnp.float32).reshape(N, H)
    hidden = _leaky(
        jnp.dot(e, w1a_ref[...], preferred_element_type=jnp.float32)
        + pooled + b1_ref[...])
    nrm = jnp.sqrt(jnp.sum(hidden * hidden, axis=1, keepdims=True))
    normalized = hidden / (nrm + 1e-6)
    out_ref[...] = (gamma_ref[...] * (normalized - mm_ref[...])
                    / jnp.sqrt(mv_ref[...] + 1e-3) + beta_ref[...])


def _post(e, pooledt, W1a, W1b, b12, gamma2, beta2, mm2, mv2):
    return pl.pallas_call(
        _post_body,
        out_shape=jax.ShapeDtypeStruct((N, H), jnp.float32),
    )(e, pooledt, W1a, W1b, b12, gamma2, beta2, mm2, mv2)


def kernel(embeddings, weights, neighbor_set, WQ, bQ, WK, bK, WV, bV, W1, b1,
           gamma, beta, moving_mean, moving_var):
    eq, evek = _project(
        embeddings, WQ, bQ.reshape(1, H), WK, bK.reshape(1, H),
        WV, bV.reshape(1, H))

    pooledt = _sc_attend(neighbor_set.reshape(N, K), eq, evek)

    out = _post(
        embeddings, pooledt, W1[:D], W1[D:], b1.reshape(1, H),
        gamma.reshape(1, H), beta.reshape(1, H),
        moving_mean.reshape(1, H), moving_var.reshape(1, H))
    return out.reshape(1, N, H)
